# Initial kernel scaffold; baseline (speedup 1.0000x reference)
import jax
import jax.numpy as jnp
from jax.experimental import pallas as pl


def kernel(edge_index, edge_weight, feat, W1, W2):
    raise NotImplementedError("write your pallas kernel here")


# speedup vs baseline: 3.5623x; 3.5623x over previous
import functools

import jax
import jax.numpy as jnp
from jax import lax
from jax.experimental import pallas as pl
from jax.experimental.pallas import tpu as pltpu
from jax.experimental.pallas import tpu_sc as plsc

L = 16
NS = 16
NCORE = 2
CHUNK = 128
HALF = 128


def _mm_body(x_ref, w_ref, o_ref, *, relu):
    x = x_ref[...]
    if relu:
        x = jnp.maximum(x, 0.0)
    o_ref[0] = jnp.dot(x, w_ref[...], preferred_element_type=jnp.float32)


def _mm_stacked(x, w, relu):
    n, fd = x.shape
    bn = n // 10
    return pl.pallas_call(
        functools.partial(_mm_body, relu=relu),
        grid=(n // bn, 2),
        in_specs=[
            pl.BlockSpec((bn, fd), lambda i, j: (i, 0)),
            pl.BlockSpec((fd, HALF), lambda i, j: (0, j)),
        ],
        out_specs=pl.BlockSpec((1, bn, HALF), lambda i, j: (j, i, 0)),
        out_shape=jax.ShapeDtypeStruct((2, n, HALF), jnp.float32),
    )(x, w)


def _spmm_sc(n_pad, xv, src2, dst_r, w_r, zrows):
    ct = dst_r.shape[1]
    rpt = n_pad // NS
    mesh = plsc.VectorSubcoreMesh(core_axis_name="c", subcore_axis_name="s")

    @functools.partial(
        pl.kernel,
        out_type=jax.ShapeDtypeStruct((n_pad, 2 * HALF), jnp.float32),
        mesh=mesh,
        scratch_types=[
            pltpu.MemorySpace.VMEM_SHARED((n_pad, HALF), jnp.float32),
            pltpu.VMEM((ct, CHUNK), jnp.int32),
            pltpu.VMEM((ct, CHUNK), jnp.int32),
            pltpu.VMEM((ct * CHUNK,), jnp.float32),
            pltpu.VMEM((CHUNK, HALF), jnp.float32),
            pltpu.SemaphoreType.DMA,
        ],
    )
    def k(xv_hbm, src_hbm, dst_hbm, w_hbm, z_hbm, out_hbm,
          acc, src_v, dst_v, w_v, rows_v, sem):
        c = lax.axis_index("c")
        s = lax.axis_index("s")
        row0 = pl.multiple_of(s * rpt, 8)
        pltpu.sync_copy(z_hbm, acc.at[pl.ds(row0, rpt)])
        pltpu.sync_copy(src_hbm.at[c, s], src_v)
        pltpu.sync_copy(dst_hbm.at[s], dst_v)
        pltpu.sync_copy(w_hbm.at[s], w_v)
        plsc.subcore_barrier()

        def chunk_body(j, carry):
            pltpu.async_copy(xv_hbm.at[src_v.at[j]], rows_v, sem).wait()

            def group_body(gi, gcarry):
                base = gi * L
                wvec = w_v[pl.ds(j * CHUNK + base, L)]
                for i in range(L):
                    wv = jnp.full((L,), wvec[i], jnp.float32)
                    e = base + i
                    for g in range(HALF // L):
                        sl = pl.ds(g * L, L)
                        rows_v[e, sl] = rows_v[e, sl] * wv
                return gcarry

            lax.fori_loop(0, CHUNK // L, group_body, 0)
            pltpu.sync_copy(rows_v, acc.at[dst_v.at[j]], add=True)
            return carry

        lax.fori_loop(0, ct, chunk_body, 0)
        plsc.subcore_barrier()
        pltpu.sync_copy(
            acc.at[pl.ds(row0, rpt)],
            out_hbm.at[pl.ds(row0, rpt), pl.ds(c * HALF, HALF)],
        )

    return k(xv, src2, dst_r, w_r, zrows)


def kernel(edge_index, edge_weight, feat, W1, W2):
    n = feat.shape[0]
    e = edge_weight.shape[0]
    n_pad = -(-n // 640) * 640
    per_tile = -(-e // (NS * CHUNK)) * CHUNK
    e_pad = per_tile * NS
    ct = per_tile // CHUNK

    dst = edge_index[0].astype(jnp.int32)
    src = edge_index[1].astype(jnp.int32)
    w = edge_weight.astype(jnp.float32)
    pad = e_pad - e
    src_p = jnp.pad(src, (0, pad))
    dst_p = jnp.pad(dst, (0, pad))
    w_p = jnp.pad(w, (0, pad))
    src2a = jnp.stack([src_p, src_p + n]).reshape(2, NS, ct, CHUNK)
    src2b = jnp.stack([src_p, src_p + n_pad]).reshape(2, NS, ct, CHUNK)
    dst_r = dst_p.reshape(NS, ct, CHUNK)
    w_r = w_p.reshape(NS, ct * CHUNK)
    zrows = jnp.zeros((n_pad // NS, HALF), jnp.float32)

    x1 = _mm_stacked(feat, W1, relu=False)
    y1 = _spmm_sc(n_pad, x1.reshape(2 * n, HALF), src2a, dst_r, w_r, zrows)
    x2 = _mm_stacked(y1, W2, relu=True)
    y2 = _spmm_sc(n_pad, x2.reshape(2 * n_pad, HALF), src2b, dst_r, w_r, zrows)
    return y2[:n]
